# baseline (speedup 1.0000x reference)
import functools

import jax
import jax.numpy as jnp
from jax.experimental import pallas as pl

N = 10000
E = 160000
T = 8
D = 128
K = 3

ROW_BLK = 400


def _gx_body(x_ref, w_ref, b_ref, o_ref):
    xb = x_ref[...]
    for g in range(3):
        acc = jnp.broadcast_to(b_ref[g][None, :], (ROW_BLK, D)).astype(jnp.float32)
        for k in range(3):
            acc = acc + jnp.dot(xb[k], w_ref[g, k],
                                preferred_element_type=jnp.float32)
        o_ref[g] = acc


def _gx_matmuls(x3, w, b):
    rows = x3.shape[1]
    grid = (rows // ROW_BLK,)
    return pl.pallas_call(
        _gx_body,
        grid=grid,
        in_specs=[
            pl.BlockSpec((3, ROW_BLK, D), lambda i: (0, i, 0)),
            pl.BlockSpec((3, 3, D, D), lambda i: (0, 0, 0, 0)),
            pl.BlockSpec((3, D), lambda i: (0, 0)),
        ],
        out_specs=pl.BlockSpec((3, ROW_BLK, D), lambda i: (0, i, 0)),
        out_shape=jax.ShapeDtypeStruct((3, rows, D), jnp.float32),
    )(x3, w, b)


def _ru_body(h_ref, h1_ref, h2_ref, gr_ref, gu_ref, wr_ref, wu_ref,
             rh_ref, u_ref):
    h = h_ref[...]
    h1 = h1_ref[...]
    h2 = h2_ref[...]
    pre_r = gr_ref[...]
    pre_u = gu_ref[...]
    basis = (h, h1, h2)
    for k in range(3):
        pre_r = pre_r + jnp.dot(basis[k], wr_ref[k],
                                preferred_element_type=jnp.float32)
        pre_u = pre_u + jnp.dot(basis[k], wu_ref[k],
                                preferred_element_type=jnp.float32)
    r = jax.nn.sigmoid(pre_r)
    rh_ref[...] = r * h
    u_ref[...] = jax.nn.sigmoid(pre_u)


def _ru_stage(h, h1, h2, gr, gu, wr, wu):
    grid = (N // ROW_BLK,)
    blk = pl.BlockSpec((ROW_BLK, D), lambda i: (i, 0))
    wblk = pl.BlockSpec((3, D, D), lambda i: (0, 0, 0))
    return pl.pallas_call(
        _ru_body,
        grid=grid,
        in_specs=[blk, blk, blk, blk, blk, wblk, wblk],
        out_specs=[blk, blk],
        out_shape=[jax.ShapeDtypeStruct((N, D), jnp.float32),
                   jax.ShapeDtypeStruct((N, D), jnp.float32)],
    )(h, h1, h2, gr, gu, wr, wu)


def _c_body(rh_ref, m1_ref, m2_ref, gm_ref, u_ref, h_ref, wm_ref, o_ref):
    pre = gm_ref[...]
    basis = (rh_ref[...], m1_ref[...], m2_ref[...])
    for k in range(3):
        pre = pre + jnp.dot(basis[k], wm_ref[k],
                            preferred_element_type=jnp.float32)
    c = jnp.tanh(pre)
    u = u_ref[...]
    o_ref[...] = u * h_ref[...] + (1.0 - u) * c


def _c_stage(rh, m1, m2, gm, u, h, wm):
    grid = (N // ROW_BLK,)
    blk = pl.BlockSpec((ROW_BLK, D), lambda i: (i, 0))
    wblk = pl.BlockSpec((3, D, D), lambda i: (0, 0, 0))
    return pl.pallas_call(
        _c_body,
        grid=grid,
        in_specs=[blk, blk, blk, blk, blk, blk, wblk],
        out_specs=blk,
        out_shape=jax.ShapeDtypeStruct((N, D), jnp.float32),
    )(rh, m1, m2, gm, u, h, wm)


def _lmul(v, src, dst, norm):
    return jnp.zeros_like(v).at[dst].add(norm[:, None] * v[src])


def kernel(x, edge_idx, edge_attr, reset_W, reset_b, update_W, update_b,
           mem_W, mem_b):
    f32 = jnp.float32
    src = edge_idx[:, 0]
    dst = edge_idx[:, 1]
    w = edge_attr

    deg = jax.vmap(lambda s, ww: jnp.zeros((N,), f32).at[s].add(ww))(src, w)
    dinv = jnp.where(deg > 0, jnp.where(deg > 0, deg, 1.0) ** -0.5, 0.0)
    norm = -jnp.take_along_axis(dinv, src, axis=1) * w * \
        jnp.take_along_axis(dinv, dst, axis=1)

    def fold(W):
        Wx, Wh = W[:, :D, :], W[:, D:, :]

        def f(Wp):
            return jnp.stack([Wp[0] - Wp[2], Wp[1], 2.0 * Wp[2]])
        return f(Wx), f(Wh)

    Wrx, Wrh = fold(reset_W)
    Wux, Wuh = fold(update_W)
    Wmx, Wmh = fold(mem_W)
    Wx_all = jnp.stack([Wrx, Wux, Wmx])
    b_all = jnp.stack([reset_b, update_b, mem_b])

    A1 = jax.vmap(_lmul)(x, src, dst, norm)
    A2 = jax.vmap(_lmul)(A1, src, dst, norm)
    x3 = jnp.stack([x, A1, A2]).reshape(3, T * N, D)
    gx = _gx_matmuls(x3, Wx_all, b_all).reshape(3, T, N, D)

    def step(h, args):
        s, d, nm, gr, gu, gm = args
        h1 = _lmul(h, s, d, nm)
        h2 = _lmul(h1, s, d, nm)
        rh, u = _ru_stage(h, h1, h2, gr, gu, Wrh, Wuh)
        m1 = _lmul(rh, s, d, nm)
        m2 = _lmul(m1, s, d, nm)
        h_new = _c_stage(rh, m1, m2, gm, u, h, Wmh)
        return h_new, h_new

    h0 = jnp.zeros((N, D), f32)
    h_fin, hs = jax.lax.scan(step, h0, (src, dst, norm, gx[0], gx[1], gx[2]))
    return (h_fin, hs)


# speedup vs baseline: 7.9894x; 7.9894x over previous
import functools

import jax
import jax.numpy as jnp
from jax import lax
from jax.experimental import pallas as pl
from jax.experimental.pallas import tpu as pltpu
from jax.experimental.pallas import tpu_sc as plsc

N = 10000
E = 160000
T = 8
D = 128
K = 3

ROW_BLK = 400

NP = 10240
EP = 163840
HALF = 64
CH = 128
RPT = NP // 16
EPT = EP // 16
NCHUNK = EPT // CH

def _make_mesh():
    return plsc.VectorSubcoreMesh(core_axis_name="c", subcore_axis_name="s",
                                  num_cores=2, num_subcores=16)

_SC_PARAMS = pltpu.CompilerParams(use_tc_tiling_on_sc=False)

ROT = 4

_GDN = jax.lax.GatherDimensionNumbers(
    offset_dims=(), collapsed_slice_dims=(0,), start_index_map=(0,))


def _splat(vec, j):
    idx = jnp.full((16, 1), j, dtype=jnp.int32)
    return lax.gather(vec, idx, _GDN, (1,),
                      mode=lax.GatherScatterMode.PROMISE_IN_BOUNDS)


def _zero_rows(rows_v):
    z = jnp.zeros((16,), jnp.float32)

    @pl.loop(0, CH)
    def _(r):
        for q in range(HALF // 16):
            rows_v[r, pl.ds(q * 16, 16)] = z


def _scale_rows(rows_v, s_v):
    @pl.loop(0, CH // 16)
    def _(g):
        svec = s_v[pl.ds(g * 16, 16)]
        for j in range(16):
            sp = _splat(svec, j)
            r = g * 16 + j
            for q in range(HALF // 16):
                sl = pl.ds(q * 16, 16)
                rows_v[r, sl] = rows_v[r, sl] * sp


def _edge_pass(e0, src_hbm, dst_hbm, w_hbm, v_sh, acc_sh, bufs):
    (src_v, dst_v, w_v, rows_v, semi, semg, sems) = bufs

    def start_idx(i, q):
        base = e0 + i * CH
        pltpu.async_copy(src_hbm.at[pl.ds(base, CH)], src_v[q], semi[q])
        pltpu.async_copy(dst_hbm.at[pl.ds(base, CH)], dst_v[q], semi[q])
        pltpu.async_copy(w_hbm.at[pl.ds(base, CH)], w_v[q], semi[q])

    def wait_idx(i, q):
        base = e0 + i * CH
        pltpu.make_async_copy(src_hbm.at[pl.ds(base, CH)], src_v[q],
                              semi[q]).wait()
        pltpu.make_async_copy(dst_hbm.at[pl.ds(base, CH)], dst_v[q],
                              semi[q]).wait()
        pltpu.make_async_copy(w_hbm.at[pl.ds(base, CH)], w_v[q],
                              semi[q]).wait()

    def start_gather(q, b):
        pltpu.async_copy(v_sh.at[src_v[q]], rows_v[b], semg[b])

    def wait_gather(q, b):
        pltpu.make_async_copy(v_sh.at[src_v[q]], rows_v[b], semg[b]).wait()

    def start_scatter(q, b):
        pltpu.async_copy(rows_v[b], acc_sh.at[dst_v[q]], sems[b], add=True)

    def wait_scatter(q, b):
        pltpu.make_async_copy(rows_v[b], acc_sh.at[dst_v[q]], sems[b]).wait()

    start_idx(0, 0)
    start_idx(1, 1)

    @pl.loop(0, NCHUNK // 4)
    def _(k):
        i0 = 4 * k
        for m in range(4):
            i = i0 + m
            q = m
            b = m % 2
            qp = (m + 3) % 4
            bp = (m + 1) % 2
            wait_idx(i, q)
            if m >= 2:
                wait_scatter(q, b)
            else:
                @pl.when(k >= 1)
                def _():
                    wait_scatter(q, b)
            start_gather(q, b)
            if m < 2:
                start_idx(i + 2, m + 2)
            else:
                @pl.when(k < NCHUNK // 4 - 1)
                def _():
                    start_idx(i + 2, m - 2)
            if m >= 1:
                wait_gather(qp, bp)
                _scale_rows(rows_v[bp], w_v[qp])
                start_scatter(qp, bp)
            else:
                @pl.when(k >= 1)
                def _():
                    wait_gather(qp, bp)
                    _scale_rows(rows_v[bp], w_v[qp])
                    start_scatter(qp, bp)

    wait_gather(3, 1)
    _scale_rows(rows_v[1], w_v[3])
    start_scatter(3, 1)
    wait_scatter(2, 0)
    wait_scatter(3, 1)


def _chain_phase(cid, r0, e0, vp, srcr, dstr, wr, d2r, out1, out2,
                 v_sh, acc_sh, bufs):
    (src_v, dst_v, w_v, rows_v, semi, semg, sems) = bufs
    pltpu.sync_copy(vp.at[cid, pl.ds(r0, RPT)], v_sh.at[pl.ds(r0, RPT)])
    _zero_rows(rows_v[0])
    for b in range(RPT // CH):
        pltpu.sync_copy(rows_v[0], acc_sh.at[pl.ds(r0 + b * CH, CH)])
    plsc.subcore_barrier()

    _edge_pass(e0, srcr, dstr, wr, v_sh, acc_sh, bufs)
    plsc.subcore_barrier()

    for b in range(RPT // CH):
        rb = r0 + b * CH
        pltpu.sync_copy(acc_sh.at[pl.ds(rb, CH)], rows_v[0])
        pltpu.sync_copy(rows_v[0], out1.at[cid, pl.ds(rb, CH)])
        pltpu.sync_copy(d2r.at[pl.ds(rb, CH)], w_v[0])
        _scale_rows(rows_v[0], w_v[0])
        pltpu.sync_copy(rows_v[0], v_sh.at[pl.ds(rb, CH)])
        _zero_rows(rows_v[0])
        pltpu.sync_copy(rows_v[0], acc_sh.at[pl.ds(rb, CH)])
    plsc.subcore_barrier()

    _edge_pass(e0, srcr, dstr, wr, v_sh, acc_sh, bufs)
    plsc.subcore_barrier()

    for b in range(RPT // CH):
        rb = r0 + b * CH
        pltpu.sync_copy(acc_sh.at[pl.ds(rb, CH)], rows_v[0])
        pltpu.sync_copy(rows_v[0], out2.at[cid, pl.ds(rb, CH)])


_SC_SCRATCH = (
    [pltpu.VMEM_SHARED((NP, HALF), jnp.float32),
     pltpu.VMEM_SHARED((NP, HALF), jnp.float32)]
    + [pltpu.VMEM((CH,), jnp.int32) for _ in range(ROT)]
    + [pltpu.VMEM((CH,), jnp.int32) for _ in range(ROT)]
    + [pltpu.VMEM((CH,), jnp.float32) for _ in range(ROT)]
    + [pltpu.VMEM((CH, HALF), jnp.float32) for _ in range(2)]
    + [pltpu.SemaphoreType.DMA for _ in range(ROT + 4)]
)


def _pack_bufs(scr):
    scr = scr[2:]
    src_v = tuple(scr[0:ROT])
    dst_v = tuple(scr[ROT:2 * ROT])
    w_v = tuple(scr[2 * ROT:3 * ROT])
    rows_v = tuple(scr[3 * ROT:3 * ROT + 2])
    semi = tuple(scr[3 * ROT + 2:4 * ROT + 2])
    semg = tuple(scr[4 * ROT + 2:4 * ROT + 4])
    sems = tuple(scr[4 * ROT + 4:4 * ROT + 6])
    return (src_v, dst_v, w_v, rows_v, semi, semg, sems)


_P_TYPE = jax.ShapeDtypeStruct((2, NP, HALF), jnp.float32)
_PT_TYPE = jax.ShapeDtypeStruct((T, 2, NP, HALF), jnp.float32)


@functools.cache
def _build_chain_kernels():
    mesh = _make_mesh()

    @functools.partial(pl.kernel, out_type=[_P_TYPE, _P_TYPE], mesh=mesh,
                       compiler_params=_SC_PARAMS, scratch_types=_SC_SCRATCH)
    def chain_one(vp, srcr, dstr, wr, d2r, out1, out2, *scr):
        cid = lax.axis_index("c")
        sid = lax.axis_index("s")
        bufs = _pack_bufs(scr)
        _chain_phase(cid, sid * RPT, sid * EPT, vp, srcr, dstr, wr, d2r,
                     out1, out2, scr[0], scr[1], bufs)

    @functools.partial(pl.kernel, out_type=[_PT_TYPE, _PT_TYPE], mesh=mesh,
                       compiler_params=_SC_PARAMS, scratch_types=_SC_SCRATCH)
    def chain_all(vp, srcr, dstr, wr, d2r, out1, out2, *scr):
        cid = lax.axis_index("c")
        sid = lax.axis_index("s")
        bufs = _pack_bufs(scr)

        @pl.loop(0, T)
        def _(t):
            _chain_phase(cid, sid * RPT, sid * EPT, vp.at[t], srcr.at[t],
                         dstr.at[t], wr.at[t], d2r.at[t], out1.at[t],
                         out2.at[t], scr[0], scr[1], bufs)
            plsc.subcore_barrier()

    return chain_one, chain_all


def _chain_sc(vp, srcr, dstr, wr, d2r):
    return _build_chain_kernels()[0](vp, srcr, dstr, wr, d2r)


def _chain_all_sc(vp, srcr, dstr, wr, d2r):
    return _build_chain_kernels()[1](vp, srcr, dstr, wr, d2r)


def _cat_scaled(p_ref, dv):
    return jnp.concatenate([p_ref[0], p_ref[1]], axis=1) * dv


def _gx_body(x_ref, a1_ref, a2_ref, dv_ref, w_ref, b_ref, o_ref):
    x = x_ref[0]
    dv = dv_ref[0]
    a1 = _cat_scaled(a1_ref[0], dv)
    a2 = _cat_scaled(a2_ref[0], dv)
    basis = (x, a1, a2)
    for g in range(3):
        acc = jnp.broadcast_to(b_ref[g][None, :], (ROW_BLK, D))
        for k in range(3):
            acc = acc + jnp.dot(basis[k], w_ref[g, k],
                                preferred_element_type=jnp.float32)
        o_ref[g, 0] = acc


def _gx_matmuls(x, a1raw, a2raw, dv, w, b):
    grid = (T, N // ROW_BLK)
    return pl.pallas_call(
        _gx_body,
        grid=grid,
        in_specs=[
            pl.BlockSpec((1, ROW_BLK, D), lambda t, i: (t, i, 0)),
            pl.BlockSpec((1, 2, ROW_BLK, HALF), lambda t, i: (t, 0, i, 0)),
            pl.BlockSpec((1, 2, ROW_BLK, HALF), lambda t, i: (t, 0, i, 0)),
            pl.BlockSpec((1, ROW_BLK, 1), lambda t, i: (t, i, 0)),
            pl.BlockSpec((3, 3, D, D), lambda t, i: (0, 0, 0, 0)),
            pl.BlockSpec((3, D), lambda t, i: (0, 0)),
        ],
        out_specs=pl.BlockSpec((3, 1, ROW_BLK, D), lambda t, i: (0, t, i, 0)),
        out_shape=jax.ShapeDtypeStruct((3, T, N, D), jnp.float32),
    )(x, a1raw, a2raw, dv, w, b)


def _ru_body(h_ref, p1_ref, p2_ref, dv_ref, gr_ref, gu_ref, wr_ref, wu_ref,
             rh_ref, rhp_ref, u_ref):
    h = h_ref[...]
    dv = dv_ref[...]
    h1 = _cat_scaled(p1_ref, dv)
    h2 = _cat_scaled(p2_ref, dv)
    pre_r = gr_ref[...]
    pre_u = gu_ref[...]
    basis = (h, h1, h2)
    for k in range(3):
        pre_r = pre_r + jnp.dot(basis[k], wr_ref[k],
                                preferred_element_type=jnp.float32)
        pre_u = pre_u + jnp.dot(basis[k], wu_ref[k],
                                preferred_element_type=jnp.float32)
    rh = jax.nn.sigmoid(pre_r) * h
    rh_ref[...] = rh
    rhp = rh * dv
    rhp_ref[0] = rhp[:, :HALF]
    rhp_ref[1] = rhp[:, HALF:]
    u_ref[...] = jax.nn.sigmoid(pre_u)


def _ru_stage(h, p1, p2, dv, gr, gu, wr, wu):
    grid = (N // ROW_BLK,)
    blk = pl.BlockSpec((ROW_BLK, D), lambda i: (i, 0))
    pblk = pl.BlockSpec((2, ROW_BLK, HALF), lambda i: (0, i, 0))
    dblk = pl.BlockSpec((ROW_BLK, 1), lambda i: (i, 0))
    wblk = pl.BlockSpec((3, D, D), lambda i: (0, 0, 0))
    return pl.pallas_call(
        _ru_body,
        grid=grid,
        in_specs=[blk, pblk, pblk, dblk, blk, blk, wblk, wblk],
        out_specs=[blk, pblk, blk],
        out_shape=[jax.ShapeDtypeStruct((N, D), jnp.float32),
                   jax.ShapeDtypeStruct((2, NP, HALF), jnp.float32),
                   jax.ShapeDtypeStruct((N, D), jnp.float32)],
    )(h, p1, p2, dv, gr, gu, wr, wu)


def _c_body(rh_ref, p1_ref, p2_ref, dv_ref, dvn_ref, gm_ref, u_ref, h_ref,
            wm_ref, h_out, hpn_out):
    dv = dv_ref[...]
    m1 = _cat_scaled(p1_ref, dv)
    m2 = _cat_scaled(p2_ref, dv)
    pre = gm_ref[...]
    basis = (rh_ref[...], m1, m2)
    for k in range(3):
        pre = pre + jnp.dot(basis[k], wm_ref[k],
                            preferred_element_type=jnp.float32)
    c = jnp.tanh(pre)
    u = u_ref[...]
    hn = u * h_ref[...] + (1.0 - u) * c
    h_out[...] = hn
    hp = hn * dvn_ref[...]
    hpn_out[0] = hp[:, :HALF]
    hpn_out[1] = hp[:, HALF:]


def _c_stage(rh, p1, p2, dv, dvn, gm, u, h, wm):
    grid = (N // ROW_BLK,)
    blk = pl.BlockSpec((ROW_BLK, D), lambda i: (i, 0))
    pblk = pl.BlockSpec((2, ROW_BLK, HALF), lambda i: (0, i, 0))
    dblk = pl.BlockSpec((ROW_BLK, 1), lambda i: (i, 0))
    wblk = pl.BlockSpec((3, D, D), lambda i: (0, 0, 0))
    return pl.pallas_call(
        _c_body,
        grid=grid,
        in_specs=[blk, pblk, pblk, dblk, dblk, blk, blk, blk, wblk],
        out_specs=[blk, pblk],
        out_shape=[jax.ShapeDtypeStruct((N, D), jnp.float32),
                   jax.ShapeDtypeStruct((2, NP, HALF), jnp.float32)],
    )(rh, p1, p2, dv, dvn, gm, u, h, wm)


def kernel(x, edge_idx, edge_attr, reset_W, reset_b, update_W, update_b,
           mem_W, mem_b):
    f32 = jnp.float32
    src = edge_idx[:, 0]
    dst = edge_idx[:, 1]
    w = edge_attr

    deg = jax.vmap(lambda s, ww: jnp.zeros((N,), f32).at[s].add(ww))(src, w)
    dinv = jnp.where(deg > 0, jnp.where(deg > 0, deg, 1.0) ** -0.5, 0.0)
    dv = dinv[:, :, None]
    dvn = jnp.roll(dv, -1, axis=0)
    d2p = jnp.pad(dinv * dinv, ((0, 0), (0, NP - N)))

    pad_idx = (jnp.arange(EP - E, dtype=jnp.int32) % N)[None, :]
    pad_idx = jnp.broadcast_to(pad_idx, (T, EP - E))
    srcp = jnp.concatenate([src, pad_idx], axis=1)
    dstp = jnp.concatenate([dst, pad_idx], axis=1)
    wneg = jnp.concatenate([-w, jnp.zeros((T, EP - E), f32)], axis=1)

    def fold(W):
        Wx, Wh = W[:, :D, :], W[:, D:, :]

        def f(Wp):
            return jnp.stack([Wp[0] - Wp[2], Wp[1], 2.0 * Wp[2]])
        return f(Wx), f(Wh)

    Wrx, Wrh = fold(reset_W)
    Wux, Wuh = fold(update_W)
    Wmx, Wmh = fold(mem_W)
    Wx_all = jnp.stack([Wrx, Wux, Wmx])
    b_all = jnp.stack([reset_b, update_b, mem_b])

    xp = x * dv
    xp = jnp.pad(xp, ((0, 0), (0, NP - N), (0, 0)))
    xp = xp.reshape(T, NP, 2, HALF).transpose(0, 2, 1, 3)
    a1raw, a2raw = _chain_all_sc(xp, srcp, dstp, wneg, d2p)
    gx = _gx_matmuls(x, a1raw, a2raw, dv, Wx_all, b_all)

    def step(carry, args):
        h, hp2 = carry
        s, d, w_t, d2_t, dv_t, dvn_t, gr, gu, gm = args
        p1h, p2h = _chain_sc(hp2, s, d, w_t, d2_t)
        rh, rhp2, u = _ru_stage(h, p1h, p2h, dv_t, gr, gu, Wrh, Wuh)
        p1m, p2m = _chain_sc(rhp2, s, d, w_t, d2_t)
        hn, hpn = _c_stage(rh, p1m, p2m, dv_t, dvn_t, gm, u, h, Wmh)
        return (hn, hpn), hn

    h0 = jnp.zeros((N, D), f32)
    hp0 = jnp.zeros((2, NP, HALF), f32)
    (h_fin, _), hs = jax.lax.scan(
        step, (h0, hp0), (srcp, dstp, wneg, d2p, dv, dvn, gx[0], gx[1], gx[2]))
    return (h_fin, hs)


# speedup vs baseline: 8.1265x; 1.0172x over previous
import functools

import jax
import jax.numpy as jnp
from jax import lax
from jax.experimental import pallas as pl
from jax.experimental.pallas import tpu as pltpu
from jax.experimental.pallas import tpu_sc as plsc

N = 10000
E = 160000
T = 8
D = 128
K = 3

ROW_BLK = 400

NP = 10240
EP = 163840
HALF = 64
CH = 128
RPT = NP // 16
EPT = EP // 16
NCHUNK = EPT // CH

def _make_mesh():
    return plsc.VectorSubcoreMesh(core_axis_name="c", subcore_axis_name="s",
                                  num_cores=2, num_subcores=16)

_SC_PARAMS = pltpu.CompilerParams(use_tc_tiling_on_sc=False)

ROT = 4

_GDN = jax.lax.GatherDimensionNumbers(
    offset_dims=(), collapsed_slice_dims=(0,), start_index_map=(0,))


def _splat(vec, j):
    idx = jnp.full((16, 1), j, dtype=jnp.int32)
    return lax.gather(vec, idx, _GDN, (1,),
                      mode=lax.GatherScatterMode.PROMISE_IN_BOUNDS)


def _zero_rows(rows_v):
    z = jnp.zeros((16,), jnp.float32)

    @pl.loop(0, CH)
    def _(r):
        for q in range(HALF // 16):
            rows_v[r, pl.ds(q * 16, 16)] = z


def _scale_rows(rows_v, s_v):
    @pl.loop(0, CH // 16)
    def _(g):
        svec = s_v[pl.ds(g * 16, 16)]
        for j in range(16):
            sp = _splat(svec, j)
            r = g * 16 + j
            for q in range(HALF // 16):
                sl = pl.ds(q * 16, 16)
                rows_v[r, sl] = rows_v[r, sl] * sp


def _edge_pass(e0, src_hbm, dst_hbm, w_hbm, vsrc, acc_sh, bufs):
    (src_v, dst_v, w_v, rows_v, semi, semg, sems) = bufs

    def start_idx(i, q):
        base = e0 + i * CH
        pltpu.async_copy(src_hbm.at[pl.ds(base, CH)], src_v[q], semi[q])
        pltpu.async_copy(dst_hbm.at[pl.ds(base, CH)], dst_v[q], semi[q])
        pltpu.async_copy(w_hbm.at[pl.ds(base, CH)], w_v[q], semi[q])

    def wait_idx(i, q):
        base = e0 + i * CH
        pltpu.make_async_copy(src_hbm.at[pl.ds(base, CH)], src_v[q],
                              semi[q]).wait()
        pltpu.make_async_copy(dst_hbm.at[pl.ds(base, CH)], dst_v[q],
                              semi[q]).wait()
        pltpu.make_async_copy(w_hbm.at[pl.ds(base, CH)], w_v[q],
                              semi[q]).wait()

    def start_gather(q, b):
        pltpu.async_copy(vsrc.at[src_v[q]], rows_v[b], semg[b])

    def wait_gather(q, b):
        pltpu.make_async_copy(vsrc.at[src_v[q]], rows_v[b], semg[b]).wait()

    def start_scatter(q, b):
        pltpu.async_copy(rows_v[b], acc_sh.at[dst_v[q]], sems[b], add=True)

    def wait_scatter(q, b):
        pltpu.make_async_copy(rows_v[b], acc_sh.at[dst_v[q]], sems[b]).wait()

    start_idx(0, 0)
    start_idx(1, 1)

    @pl.loop(0, NCHUNK // 4)
    def _(k):
        i0 = 4 * k
        for m in range(4):
            i = i0 + m
            q = m
            b = m % 2
            qp = (m + 3) % 4
            bp = (m + 1) % 2
            wait_idx(i, q)
            if m >= 2:
                wait_scatter(q, b)
            else:
                @pl.when(k >= 1)
                def _():
                    wait_scatter(q, b)
            start_gather(q, b)
            if m < 2:
                start_idx(i + 2, m + 2)
            else:
                @pl.when(k < NCHUNK // 4 - 1)
                def _():
                    start_idx(i + 2, m - 2)
            if m >= 1:
                wait_gather(qp, bp)
                _scale_rows(rows_v[bp], w_v[qp])
                start_scatter(qp, bp)
            else:
                @pl.when(k >= 1)
                def _():
                    wait_gather(qp, bp)
                    _scale_rows(rows_v[bp], w_v[qp])
                    start_scatter(qp, bp)

    wait_gather(3, 1)
    _scale_rows(rows_v[1], w_v[3])
    start_scatter(3, 1)
    wait_scatter(2, 0)
    wait_scatter(3, 1)


def _chain_phase(cid, r0, e0, vp, srcr, dstr, wr, d2r, out1, out2, qout,
                 acc_sh, bufs):
    (src_v, dst_v, w_v, rows_v, semi, semg, sems) = bufs
    _zero_rows(rows_v[0])
    for b in range(RPT // CH):
        pltpu.sync_copy(rows_v[0], acc_sh.at[pl.ds(r0 + b * CH, CH)])
    plsc.subcore_barrier()

    _edge_pass(e0, srcr, dstr, wr, vp.at[cid], acc_sh, bufs)
    plsc.subcore_barrier()

    for b in range(RPT // CH):
        rb = r0 + b * CH
        pltpu.sync_copy(acc_sh.at[pl.ds(rb, CH)], rows_v[0])
        pltpu.sync_copy(rows_v[0], out1.at[cid, pl.ds(rb, CH)])
        pltpu.sync_copy(d2r.at[pl.ds(rb, CH)], w_v[0])
        _scale_rows(rows_v[0], w_v[0])
        pltpu.sync_copy(rows_v[0], qout.at[cid, pl.ds(rb, CH)])
        _zero_rows(rows_v[0])
        pltpu.sync_copy(rows_v[0], acc_sh.at[pl.ds(rb, CH)])
    plsc.subcore_barrier()

    _edge_pass(e0, srcr, dstr, wr, qout.at[cid], acc_sh, bufs)
    plsc.subcore_barrier()

    for b in range(RPT // CH):
        rb = r0 + b * CH
        pltpu.sync_copy(acc_sh.at[pl.ds(rb, CH)], rows_v[0])
        pltpu.sync_copy(rows_v[0], out2.at[cid, pl.ds(rb, CH)])


_SC_SCRATCH = (
    [pltpu.VMEM_SHARED((NP, HALF), jnp.float32)]
    + [pltpu.VMEM((CH,), jnp.int32) for _ in range(ROT)]
    + [pltpu.VMEM((CH,), jnp.int32) for _ in range(ROT)]
    + [pltpu.VMEM((CH,), jnp.float32) for _ in range(ROT)]
    + [pltpu.VMEM((CH, HALF), jnp.float32) for _ in range(2)]
    + [pltpu.SemaphoreType.DMA for _ in range(ROT + 4)]
)


def _pack_bufs(scr):
    scr = scr[1:]
    src_v = tuple(scr[0:ROT])
    dst_v = tuple(scr[ROT:2 * ROT])
    w_v = tuple(scr[2 * ROT:3 * ROT])
    rows_v = tuple(scr[3 * ROT:3 * ROT + 2])
    semi = tuple(scr[3 * ROT + 2:4 * ROT + 2])
    semg = tuple(scr[4 * ROT + 2:4 * ROT + 4])
    sems = tuple(scr[4 * ROT + 4:4 * ROT + 6])
    return (src_v, dst_v, w_v, rows_v, semi, semg, sems)


_P_TYPE = jax.ShapeDtypeStruct((2, NP, HALF), jnp.float32)
_PT_TYPE = jax.ShapeDtypeStruct((T, 2, NP, HALF), jnp.float32)


@functools.cache
def _build_chain_kernels():
    mesh = _make_mesh()

    @functools.partial(pl.kernel,
                       out_type=[_P_TYPE, _P_TYPE, _P_TYPE], mesh=mesh,
                       compiler_params=_SC_PARAMS, scratch_types=_SC_SCRATCH)
    def chain_one(vp, srcr, dstr, wr, d2r, out1, out2, qout, *scr):
        cid = lax.axis_index("c")
        sid = lax.axis_index("s")
        bufs = _pack_bufs(scr)
        _chain_phase(cid, sid * RPT, sid * EPT, vp, srcr, dstr, wr, d2r,
                     out1, out2, qout, scr[0], bufs)

    @functools.partial(pl.kernel,
                       out_type=[_PT_TYPE, _PT_TYPE, _P_TYPE], mesh=mesh,
                       compiler_params=_SC_PARAMS, scratch_types=_SC_SCRATCH)
    def chain_all(vp, srcr, dstr, wr, d2r, out1, out2, qout, *scr):
        cid = lax.axis_index("c")
        sid = lax.axis_index("s")
        bufs = _pack_bufs(scr)

        @pl.loop(0, T)
        def _(t):
            _chain_phase(cid, sid * RPT, sid * EPT, vp.at[t], srcr.at[t],
                         dstr.at[t], wr.at[t], d2r.at[t], out1.at[t],
                         out2.at[t], qout, scr[0], bufs)
            plsc.subcore_barrier()

    return chain_one, chain_all


def _chain_sc(vp, srcr, dstr, wr, d2r):
    return _build_chain_kernels()[0](vp, srcr, dstr, wr, d2r)[:2]


def _chain_all_sc(vp, srcr, dstr, wr, d2r):
    return _build_chain_kernels()[1](vp, srcr, dstr, wr, d2r)[:2]


def _cat_scaled(p_ref, dv):
    return jnp.concatenate([p_ref[0], p_ref[1]], axis=1) * dv


def _gx_body(x_ref, a1_ref, a2_ref, dv_ref, w_ref, b_ref, o_ref):
    x = x_ref[0]
    dv = dv_ref[0]
    a1 = _cat_scaled(a1_ref[0], dv)
    a2 = _cat_scaled(a2_ref[0], dv)
    basis = (x, a1, a2)
    for g in range(3):
        acc = jnp.broadcast_to(b_ref[g][None, :], (ROW_BLK, D))
        for k in range(3):
            acc = acc + jnp.dot(basis[k], w_ref[g, k],
                                preferred_element_type=jnp.float32)
        o_ref[g, 0] = acc


def _gx_matmuls(x, a1raw, a2raw, dv, w, b):
    grid = (T, N // ROW_BLK)
    return pl.pallas_call(
        _gx_body,
        grid=grid,
        in_specs=[
            pl.BlockSpec((1, ROW_BLK, D), lambda t, i: (t, i, 0)),
            pl.BlockSpec((1, 2, ROW_BLK, HALF), lambda t, i: (t, 0, i, 0)),
            pl.BlockSpec((1, 2, ROW_BLK, HALF), lambda t, i: (t, 0, i, 0)),
            pl.BlockSpec((1, ROW_BLK, 1), lambda t, i: (t, i, 0)),
            pl.BlockSpec((3, 3, D, D), lambda t, i: (0, 0, 0, 0)),
            pl.BlockSpec((3, D), lambda t, i: (0, 0)),
        ],
        out_specs=pl.BlockSpec((3, 1, ROW_BLK, D), lambda t, i: (0, t, i, 0)),
        out_shape=jax.ShapeDtypeStruct((3, T, N, D), jnp.float32),
    )(x, a1raw, a2raw, dv, w, b)


def _ru_body(h_ref, p1_ref, p2_ref, dv_ref, gr_ref, gu_ref, wr_ref, wu_ref,
             rh_ref, rhp_ref, u_ref):
    h = h_ref[...]
    dv = dv_ref[...]
    h1 = _cat_scaled(p1_ref, dv)
    h2 = _cat_scaled(p2_ref, dv)
    pre_r = gr_ref[...]
    pre_u = gu_ref[...]
    basis = (h, h1, h2)
    for k in range(3):
        pre_r = pre_r + jnp.dot(basis[k], wr_ref[k],
                                preferred_element_type=jnp.float32)
        pre_u = pre_u + jnp.dot(basis[k], wu_ref[k],
                                preferred_element_type=jnp.float32)
    rh = jax.nn.sigmoid(pre_r) * h
    rh_ref[...] = rh
    rhp = rh * dv
    rhp_ref[0] = rhp[:, :HALF]
    rhp_ref[1] = rhp[:, HALF:]
    u_ref[...] = jax.nn.sigmoid(pre_u)


def _ru_stage(h, p1, p2, dv, gr, gu, wr, wu):
    grid = (N // ROW_BLK,)
    blk = pl.BlockSpec((ROW_BLK, D), lambda i: (i, 0))
    pblk = pl.BlockSpec((2, ROW_BLK, HALF), lambda i: (0, i, 0))
    dblk = pl.BlockSpec((ROW_BLK, 1), lambda i: (i, 0))
    wblk = pl.BlockSpec((3, D, D), lambda i: (0, 0, 0))
    return pl.pallas_call(
        _ru_body,
        grid=grid,
        in_specs=[blk, pblk, pblk, dblk, blk, blk, wblk, wblk],
        out_specs=[blk, pblk, blk],
        out_shape=[jax.ShapeDtypeStruct((N, D), jnp.float32),
                   jax.ShapeDtypeStruct((2, NP, HALF), jnp.float32),
                   jax.ShapeDtypeStruct((N, D), jnp.float32)],
    )(h, p1, p2, dv, gr, gu, wr, wu)


def _c_body(rh_ref, p1_ref, p2_ref, dv_ref, dvn_ref, gm_ref, u_ref, h_ref,
            wm_ref, h_out, hpn_out):
    dv = dv_ref[...]
    m1 = _cat_scaled(p1_ref, dv)
    m2 = _cat_scaled(p2_ref, dv)
    pre = gm_ref[...]
    basis = (rh_ref[...], m1, m2)
    for k in range(3):
        pre = pre + jnp.dot(basis[k], wm_ref[k],
                            preferred_element_type=jnp.float32)
    c = jnp.tanh(pre)
    u = u_ref[...]
    hn = u * h_ref[...] + (1.0 - u) * c
    h_out[...] = hn
    hp = hn * dvn_ref[...]
    hpn_out[0] = hp[:, :HALF]
    hpn_out[1] = hp[:, HALF:]


def _c_stage(rh, p1, p2, dv, dvn, gm, u, h, wm):
    grid = (N // ROW_BLK,)
    blk = pl.BlockSpec((ROW_BLK, D), lambda i: (i, 0))
    pblk = pl.BlockSpec((2, ROW_BLK, HALF), lambda i: (0, i, 0))
    dblk = pl.BlockSpec((ROW_BLK, 1), lambda i: (i, 0))
    wblk = pl.BlockSpec((3, D, D), lambda i: (0, 0, 0))
    return pl.pallas_call(
        _c_body,
        grid=grid,
        in_specs=[blk, pblk, pblk, dblk, dblk, blk, blk, blk, wblk],
        out_specs=[blk, pblk],
        out_shape=[jax.ShapeDtypeStruct((N, D), jnp.float32),
                   jax.ShapeDtypeStruct((2, NP, HALF), jnp.float32)],
    )(rh, p1, p2, dv, dvn, gm, u, h, wm)


def kernel(x, edge_idx, edge_attr, reset_W, reset_b, update_W, update_b,
           mem_W, mem_b):
    f32 = jnp.float32
    src = edge_idx[:, 0]
    dst = edge_idx[:, 1]
    w = edge_attr

    deg = jax.vmap(lambda s, ww: jnp.zeros((N,), f32).at[s].add(ww))(src, w)
    dinv = jnp.where(deg > 0, jnp.where(deg > 0, deg, 1.0) ** -0.5, 0.0)
    dv = dinv[:, :, None]
    dvn = jnp.roll(dv, -1, axis=0)
    d2p = jnp.pad(dinv * dinv, ((0, 0), (0, NP - N)))

    pad_idx = (jnp.arange(EP - E, dtype=jnp.int32) % N)[None, :]
    pad_idx = jnp.broadcast_to(pad_idx, (T, EP - E))
    srcp = jnp.concatenate([src, pad_idx], axis=1)
    dstp = jnp.concatenate([dst, pad_idx], axis=1)
    wneg = jnp.concatenate([-w, jnp.zeros((T, EP - E), f32)], axis=1)

    def fold(W):
        Wx, Wh = W[:, :D, :], W[:, D:, :]

        def f(Wp):
            return jnp.stack([Wp[0] - Wp[2], Wp[1], 2.0 * Wp[2]])
        return f(Wx), f(Wh)

    Wrx, Wrh = fold(reset_W)
    Wux, Wuh = fold(update_W)
    Wmx, Wmh = fold(mem_W)
    Wx_all = jnp.stack([Wrx, Wux, Wmx])
    b_all = jnp.stack([reset_b, update_b, mem_b])

    xp = x * dv
    xp = jnp.pad(xp, ((0, 0), (0, NP - N), (0, 0)))
    xp = xp.reshape(T, NP, 2, HALF).transpose(0, 2, 1, 3)
    a1raw, a2raw = _chain_all_sc(xp, srcp, dstp, wneg, d2p)
    gx = _gx_matmuls(x, a1raw, a2raw, dv, Wx_all, b_all)

    def step(carry, args):
        h, hp2 = carry
        s, d, w_t, d2_t, dv_t, dvn_t, gr, gu, gm = args
        p1h, p2h = _chain_sc(hp2, s, d, w_t, d2_t)
        rh, rhp2, u = _ru_stage(h, p1h, p2h, dv_t, gr, gu, Wrh, Wuh)
        p1m, p2m = _chain_sc(rhp2, s, d, w_t, d2_t)
        hn, hpn = _c_stage(rh, p1m, p2m, dv_t, dvn_t, gm, u, h, Wmh)
        return (hn, hpn), hn

    h0 = jnp.zeros((N, D), f32)
    hp0 = jnp.zeros((2, NP, HALF), f32)
    (h_fin, _), hs = jax.lax.scan(
        step, (h0, hp0), (srcp, dstp, wneg, d2p, dv, dvn, gx[0], gx[1], gx[2]))
    return (h_fin, hs)


# speedup vs baseline: 8.9465x; 1.1009x over previous
import functools

import jax
import jax.numpy as jnp
from jax import lax
from jax.experimental import pallas as pl
from jax.experimental.pallas import tpu as pltpu
from jax.experimental.pallas import tpu_sc as plsc

N = 10000
E = 160000
T = 8
D = 128
K = 3

ROW_BLK = 400

NP = 10240
EP = 163840
HALF = 64
CH = 128
RPT = NP // 16
EPT = EP // 16
NCHUNK = EPT // CH

def _make_mesh():
    return plsc.VectorSubcoreMesh(core_axis_name="c", subcore_axis_name="s",
                                  num_cores=2, num_subcores=16)

_SC_PARAMS = pltpu.CompilerParams(use_tc_tiling_on_sc=False)

ROT = 4

_GDN = jax.lax.GatherDimensionNumbers(
    offset_dims=(), collapsed_slice_dims=(0,), start_index_map=(0,))


def _splat(vec, j):
    idx = jnp.full((16, 1), j, dtype=jnp.int32)
    return lax.gather(vec, idx, _GDN, (1,),
                      mode=lax.GatherScatterMode.PROMISE_IN_BOUNDS)


def _zero_rows(rows_v):
    z = jnp.zeros((16,), jnp.float32)

    @pl.loop(0, CH)
    def _(r):
        for q in range(HALF // 16):
            rows_v[r, pl.ds(q * 16, 16)] = z


def _scale_rows(rows_v, s_v):
    @pl.loop(0, CH // 16)
    def _(g):
        svec = s_v[pl.ds(g * 16, 16)]
        for j in range(16):
            sp = _splat(svec, j)
            r = g * 16 + j
            for q in range(HALF // 16):
                sl = pl.ds(q * 16, 16)
                rows_v[r, sl] = rows_v[r, sl] * sp


def _edge_pass(e0, src_hbm, dst_hbm, w_hbm, vsrc, acc_sh, bufs):
    (src_v, dst_v, w_v, rows_v, semi, semg, sems) = bufs

    def start_idx(i, q):
        base = e0 + i * CH
        pltpu.async_copy(src_hbm.at[pl.ds(base, CH)], src_v[q], semi[q])
        pltpu.async_copy(dst_hbm.at[pl.ds(base, CH)], dst_v[q], semi[q])
        pltpu.async_copy(w_hbm.at[pl.ds(base, CH)], w_v[q], semi[q])

    def wait_idx(i, q):
        base = e0 + i * CH
        pltpu.make_async_copy(src_hbm.at[pl.ds(base, CH)], src_v[q],
                              semi[q]).wait()
        pltpu.make_async_copy(dst_hbm.at[pl.ds(base, CH)], dst_v[q],
                              semi[q]).wait()
        pltpu.make_async_copy(w_hbm.at[pl.ds(base, CH)], w_v[q],
                              semi[q]).wait()

    def start_gather(q, r):
        pltpu.async_copy(vsrc.at[src_v[q]], rows_v[r], semg[r])

    def wait_gather(q, r):
        pltpu.make_async_copy(vsrc.at[src_v[q]], rows_v[r], semg[r]).wait()

    def start_scatter(q, r):
        pltpu.async_copy(rows_v[r], acc_sh.at[dst_v[q]], sems[r], add=True)

    def wait_scatter(q, r):
        pltpu.make_async_copy(rows_v[r], acc_sh.at[dst_v[q]], sems[r]).wait()

    for c in range(4):
        start_idx(c, c)

    nouter = NCHUNK // 8

    @pl.loop(0, nouter)
    def _(k):
        i0 = 8 * k
        for m in range(8):
            i = i0 + m
            q = m
            r = m % 4
            qp = (m + 6) % 8
            rp = (m + 2) % 4
            wait_idx(i, q)
            if m >= 4:
                wait_scatter(q, r)
            else:
                @pl.when(k >= 1)
                def _():
                    wait_scatter(q, r)
            start_gather(q, r)
            if m >= 2:
                wait_gather(qp, rp)
                _scale_rows(rows_v[rp], w_v[qp])
                start_scatter(qp, rp)
            else:
                @pl.when(k >= 1)
                def _():
                    wait_gather(qp, rp)
                    _scale_rows(rows_v[rp], w_v[qp])
                    start_scatter(qp, rp)
            if m < 4:
                start_idx(i + 4, m + 4)
            else:
                @pl.when(k < nouter - 1)
                def _():
                    start_idx(i + 4, m - 4)

    for i in (NCHUNK - 2, NCHUNK - 1):
        wait_gather(i % 8, i % 4)
        _scale_rows(rows_v[i % 4], w_v[i % 8])
        start_scatter(i % 8, i % 4)
    for i in range(NCHUNK - 4, NCHUNK):
        wait_scatter(i % 8, i % 4)


def _chain_phase(cid, r0, e0, vp, srcr, dstr, wr, d2r, out1, out2, qout,
                 acc_sh, bufs):
    (src_v, dst_v, w_v, rows_v, semi, semg, sems) = bufs
    _zero_rows(rows_v[0])
    for b in range(RPT // CH):
        pltpu.sync_copy(rows_v[0], acc_sh.at[pl.ds(r0 + b * CH, CH)])
    plsc.subcore_barrier()

    _edge_pass(e0, srcr, dstr, wr, vp.at[cid], acc_sh, bufs)
    plsc.subcore_barrier()

    for b in range(RPT // CH):
        rb = r0 + b * CH
        pltpu.sync_copy(acc_sh.at[pl.ds(rb, CH)], rows_v[0])
        pltpu.sync_copy(rows_v[0], out1.at[cid, pl.ds(rb, CH)])
        pltpu.sync_copy(d2r.at[pl.ds(rb, CH)], w_v[0])
        _scale_rows(rows_v[0], w_v[0])
        pltpu.sync_copy(rows_v[0], qout.at[cid, pl.ds(rb, CH)])
        _zero_rows(rows_v[0])
        pltpu.sync_copy(rows_v[0], acc_sh.at[pl.ds(rb, CH)])
    plsc.subcore_barrier()

    _edge_pass(e0, srcr, dstr, wr, qout.at[cid], acc_sh, bufs)
    plsc.subcore_barrier()

    for b in range(RPT // CH):
        rb = r0 + b * CH
        pltpu.sync_copy(acc_sh.at[pl.ds(rb, CH)], rows_v[0])
        pltpu.sync_copy(rows_v[0], out2.at[cid, pl.ds(rb, CH)])


NIDX = 8
NROW = 4

_SC_SCRATCH = (
    [pltpu.VMEM_SHARED((NP, HALF), jnp.float32)]
    + [pltpu.VMEM((CH,), jnp.int32) for _ in range(NIDX)]
    + [pltpu.VMEM((CH,), jnp.int32) for _ in range(NIDX)]
    + [pltpu.VMEM((CH,), jnp.float32) for _ in range(NIDX)]
    + [pltpu.VMEM((CH, HALF), jnp.float32) for _ in range(NROW)]
    + [pltpu.SemaphoreType.DMA for _ in range(NIDX + 2 * NROW)]
)


def _pack_bufs(scr):
    scr = scr[1:]
    src_v = tuple(scr[0:NIDX])
    dst_v = tuple(scr[NIDX:2 * NIDX])
    w_v = tuple(scr[2 * NIDX:3 * NIDX])
    rows_v = tuple(scr[3 * NIDX:3 * NIDX + NROW])
    sems_all = scr[3 * NIDX + NROW:]
    semi = tuple(sems_all[0:NIDX])
    semg = tuple(sems_all[NIDX:NIDX + NROW])
    sems = tuple(sems_all[NIDX + NROW:NIDX + 2 * NROW])
    return (src_v, dst_v, w_v, rows_v, semi, semg, sems)


_P_TYPE = jax.ShapeDtypeStruct((2, NP, HALF), jnp.float32)
_PT_TYPE = jax.ShapeDtypeStruct((T, 2, NP, HALF), jnp.float32)


@functools.cache
def _build_chain_kernels():
    mesh = _make_mesh()

    @functools.partial(pl.kernel,
                       out_type=[_P_TYPE, _P_TYPE, _P_TYPE], mesh=mesh,
                       compiler_params=_SC_PARAMS, scratch_types=_SC_SCRATCH)
    def chain_one(vp, srcr, dstr, wr, d2r, out1, out2, qout, *scr):
        cid = lax.axis_index("c")
        sid = lax.axis_index("s")
        bufs = _pack_bufs(scr)
        _chain_phase(cid, sid * RPT, sid * EPT, vp, srcr, dstr, wr, d2r,
                     out1, out2, qout, scr[0], bufs)

    @functools.partial(pl.kernel,
                       out_type=[_PT_TYPE, _PT_TYPE, _P_TYPE], mesh=mesh,
                       compiler_params=_SC_PARAMS, scratch_types=_SC_SCRATCH)
    def chain_all(vp, srcr, dstr, wr, d2r, out1, out2, qout, *scr):
        cid = lax.axis_index("c")
        sid = lax.axis_index("s")
        bufs = _pack_bufs(scr)

        @pl.loop(0, T)
        def _(t):
            _chain_phase(cid, sid * RPT, sid * EPT, vp.at[t], srcr.at[t],
                         dstr.at[t], wr.at[t], d2r.at[t], out1.at[t],
                         out2.at[t], qout, scr[0], bufs)
            plsc.subcore_barrier()

    return chain_one, chain_all


def _chain_sc(vp, srcr, dstr, wr, d2r):
    return _build_chain_kernels()[0](vp, srcr, dstr, wr, d2r)[:2]


def _chain_all_sc(vp, srcr, dstr, wr, d2r):
    return _build_chain_kernels()[1](vp, srcr, dstr, wr, d2r)[:2]


def _cat_scaled(p_ref, dv):
    return jnp.concatenate([p_ref[0], p_ref[1]], axis=1) * dv


def _gx_body(x_ref, a1_ref, a2_ref, dv_ref, w_ref, b_ref, o_ref):
    x = x_ref[0]
    dv = dv_ref[0]
    a1 = _cat_scaled(a1_ref[0], dv)
    a2 = _cat_scaled(a2_ref[0], dv)
    basis = (x, a1, a2)
    for g in range(3):
        acc = jnp.broadcast_to(b_ref[g][None, :], (ROW_BLK, D))
        for k in range(3):
            acc = acc + jnp.dot(basis[k], w_ref[g, k],
                                preferred_element_type=jnp.float32)
        o_ref[g, 0] = acc


def _gx_matmuls(x, a1raw, a2raw, dv, w, b):
    grid = (T, N // ROW_BLK)
    return pl.pallas_call(
        _gx_body,
        grid=grid,
        in_specs=[
            pl.BlockSpec((1, ROW_BLK, D), lambda t, i: (t, i, 0)),
            pl.BlockSpec((1, 2, ROW_BLK, HALF), lambda t, i: (t, 0, i, 0)),
            pl.BlockSpec((1, 2, ROW_BLK, HALF), lambda t, i: (t, 0, i, 0)),
            pl.BlockSpec((1, ROW_BLK, 1), lambda t, i: (t, i, 0)),
            pl.BlockSpec((3, 3, D, D), lambda t, i: (0, 0, 0, 0)),
            pl.BlockSpec((3, D), lambda t, i: (0, 0)),
        ],
        out_specs=pl.BlockSpec((3, 1, ROW_BLK, D), lambda t, i: (0, t, i, 0)),
        out_shape=jax.ShapeDtypeStruct((3, T, N, D), jnp.float32),
    )(x, a1raw, a2raw, dv, w, b)


def _ru_body(h_ref, p1_ref, p2_ref, dv_ref, gr_ref, gu_ref, wr_ref, wu_ref,
             rh_ref, rhp_ref, u_ref):
    h = h_ref[...]
    dv = dv_ref[...]
    h1 = _cat_scaled(p1_ref, dv)
    h2 = _cat_scaled(p2_ref, dv)
    pre_r = gr_ref[...]
    pre_u = gu_ref[...]
    basis = (h, h1, h2)
    for k in range(3):
        pre_r = pre_r + jnp.dot(basis[k], wr_ref[k],
                                preferred_element_type=jnp.float32)
        pre_u = pre_u + jnp.dot(basis[k], wu_ref[k],
                                preferred_element_type=jnp.float32)
    rh = jax.nn.sigmoid(pre_r) * h
    rh_ref[...] = rh
    rhp = rh * dv
    rhp_ref[0] = rhp[:, :HALF]
    rhp_ref[1] = rhp[:, HALF:]
    u_ref[...] = jax.nn.sigmoid(pre_u)


def _ru_stage(h, p1, p2, dv, gr, gu, wr, wu):
    grid = (N // ROW_BLK,)
    blk = pl.BlockSpec((ROW_BLK, D), lambda i: (i, 0))
    pblk = pl.BlockSpec((2, ROW_BLK, HALF), lambda i: (0, i, 0))
    dblk = pl.BlockSpec((ROW_BLK, 1), lambda i: (i, 0))
    wblk = pl.BlockSpec((3, D, D), lambda i: (0, 0, 0))
    return pl.pallas_call(
        _ru_body,
        grid=grid,
        in_specs=[blk, pblk, pblk, dblk, blk, blk, wblk, wblk],
        out_specs=[blk, pblk, blk],
        out_shape=[jax.ShapeDtypeStruct((N, D), jnp.float32),
                   jax.ShapeDtypeStruct((2, NP, HALF), jnp.float32),
                   jax.ShapeDtypeStruct((N, D), jnp.float32)],
    )(h, p1, p2, dv, gr, gu, wr, wu)


def _c_body(rh_ref, p1_ref, p2_ref, dv_ref, dvn_ref, gm_ref, u_ref, h_ref,
            wm_ref, h_out, hpn_out):
    dv = dv_ref[...]
    m1 = _cat_scaled(p1_ref, dv)
    m2 = _cat_scaled(p2_ref, dv)
    pre = gm_ref[...]
    basis = (rh_ref[...], m1, m2)
    for k in range(3):
        pre = pre + jnp.dot(basis[k], wm_ref[k],
                            preferred_element_type=jnp.float32)
    c = jnp.tanh(pre)
    u = u_ref[...]
    hn = u * h_ref[...] + (1.0 - u) * c
    h_out[...] = hn
    hp = hn * dvn_ref[...]
    hpn_out[0] = hp[:, :HALF]
    hpn_out[1] = hp[:, HALF:]


def _c_stage(rh, p1, p2, dv, dvn, gm, u, h, wm):
    grid = (N // ROW_BLK,)
    blk = pl.BlockSpec((ROW_BLK, D), lambda i: (i, 0))
    pblk = pl.BlockSpec((2, ROW_BLK, HALF), lambda i: (0, i, 0))
    dblk = pl.BlockSpec((ROW_BLK, 1), lambda i: (i, 0))
    wblk = pl.BlockSpec((3, D, D), lambda i: (0, 0, 0))
    return pl.pallas_call(
        _c_body,
        grid=grid,
        in_specs=[blk, pblk, pblk, dblk, dblk, blk, blk, blk, wblk],
        out_specs=[blk, pblk],
        out_shape=[jax.ShapeDtypeStruct((N, D), jnp.float32),
                   jax.ShapeDtypeStruct((2, NP, HALF), jnp.float32)],
    )(rh, p1, p2, dv, dvn, gm, u, h, wm)


def kernel(x, edge_idx, edge_attr, reset_W, reset_b, update_W, update_b,
           mem_W, mem_b):
    f32 = jnp.float32
    src = edge_idx[:, 0]
    dst = edge_idx[:, 1]
    w = edge_attr

    deg = jax.vmap(lambda s, ww: jnp.zeros((N,), f32).at[s].add(ww))(src, w)
    dinv = jnp.where(deg > 0, jnp.where(deg > 0, deg, 1.0) ** -0.5, 0.0)
    dv = dinv[:, :, None]
    dvn = jnp.roll(dv, -1, axis=0)
    d2p = jnp.pad(dinv * dinv, ((0, 0), (0, NP - N)))

    pad_idx = (jnp.arange(EP - E, dtype=jnp.int32) % N)[None, :]
    pad_idx = jnp.broadcast_to(pad_idx, (T, EP - E))
    srcp = jnp.concatenate([src, pad_idx], axis=1)
    dstp = jnp.concatenate([dst, pad_idx], axis=1)
    wneg = jnp.concatenate([-w, jnp.zeros((T, EP - E), f32)], axis=1)

    def fold(W):
        Wx, Wh = W[:, :D, :], W[:, D:, :]

        def f(Wp):
            return jnp.stack([Wp[0] - Wp[2], Wp[1], 2.0 * Wp[2]])
        return f(Wx), f(Wh)

    Wrx, Wrh = fold(reset_W)
    Wux, Wuh = fold(update_W)
    Wmx, Wmh = fold(mem_W)
    Wx_all = jnp.stack([Wrx, Wux, Wmx])
    b_all = jnp.stack([reset_b, update_b, mem_b])

    xp = x * dv
    xp = jnp.pad(xp, ((0, 0), (0, NP - N), (0, 0)))
    xp = xp.reshape(T, NP, 2, HALF).transpose(0, 2, 1, 3)
    a1raw, a2raw = _chain_all_sc(xp, srcp, dstp, wneg, d2p)
    gx = _gx_matmuls(x, a1raw, a2raw, dv, Wx_all, b_all)

    def step(carry, args):
        h, hp2 = carry
        s, d, w_t, d2_t, dv_t, dvn_t, gr, gu, gm = args
        p1h, p2h = _chain_sc(hp2, s, d, w_t, d2_t)
        rh, rhp2, u = _ru_stage(h, p1h, p2h, dv_t, gr, gu, Wrh, Wuh)
        p1m, p2m = _chain_sc(rhp2, s, d, w_t, d2_t)
        hn, hpn = _c_stage(rh, p1m, p2m, dv_t, dvn_t, gm, u, h, Wmh)
        return (hn, hpn), hn

    h0 = jnp.zeros((N, D), f32)
    hp0 = jnp.zeros((2, NP, HALF), f32)
    (h_fin, _), hs = jax.lax.scan(
        step, (h0, hp0), (srcp, dstp, wneg, d2p, dv, dvn, gx[0], gx[1], gx[2]))
    return (h_fin, hs)


# speedup vs baseline: 17.1161x; 1.9132x over previous
import functools

import jax
import jax.numpy as jnp
from jax import lax
from jax.experimental import pallas as pl
from jax.experimental.pallas import tpu as pltpu
from jax.experimental.pallas import tpu_sc as plsc

N = 10000
E = 160000
T = 8
D = 128
K = 3

ROW_BLK = 400

NP = 10240
EP = 163840
HALF = 64
CH = 128
RPT = NP // 16
EPT = EP // 16
NCHUNK = EPT // CH

def _make_mesh():
    return plsc.VectorSubcoreMesh(core_axis_name="c", subcore_axis_name="s",
                                  num_cores=2, num_subcores=16)

_SC_PARAMS = pltpu.CompilerParams(use_tc_tiling_on_sc=False)

ROT = 4

_GDN = jax.lax.GatherDimensionNumbers(
    offset_dims=(), collapsed_slice_dims=(0,), start_index_map=(0,))


def _splat(vec, j):
    idx = jnp.full((16, 1), j, dtype=jnp.int32)
    return lax.gather(vec, idx, _GDN, (1,),
                      mode=lax.GatherScatterMode.PROMISE_IN_BOUNDS)


def _zero_rows(rows_v):
    z = jnp.zeros((16,), jnp.float32)

    @pl.loop(0, CH)
    def _(r):
        for q in range(HALF // 16):
            rows_v[r, pl.ds(q * 16, 16)] = z


def _scale_rows(rows_v, s_v):
    @plsc.parallel_loop(0, CH // 16, unroll=2)
    def _(g):
        svec = s_v[pl.ds(g * 16, 16)]
        for j in range(16):
            sp = _splat(svec, j)
            r = g * 16 + j
            for q in range(HALF // 16):
                sl = pl.ds(q * 16, 16)
                rows_v[r, sl] = rows_v[r, sl] * sp


def _edge_pass(e0, src_hbm, dst_hbm, w_hbm, vsrc, acc_sh, bufs):
    (src_v, dst_v, w_v, rows_v, semi, semg, sems) = bufs

    def start_idx(i, q):
        base = e0 + i * CH
        pltpu.async_copy(src_hbm.at[pl.ds(base, CH)], src_v[q], semi[q])
        pltpu.async_copy(dst_hbm.at[pl.ds(base, CH)], dst_v[q], semi[q])
        pltpu.async_copy(w_hbm.at[pl.ds(base, CH)], w_v[q], semi[q])

    def wait_idx(i, q):
        base = e0 + i * CH
        pltpu.make_async_copy(src_hbm.at[pl.ds(base, CH)], src_v[q],
                              semi[q]).wait()
        pltpu.make_async_copy(dst_hbm.at[pl.ds(base, CH)], dst_v[q],
                              semi[q]).wait()
        pltpu.make_async_copy(w_hbm.at[pl.ds(base, CH)], w_v[q],
                              semi[q]).wait()

    def start_gather(q, r):
        pltpu.async_copy(vsrc.at[src_v[q]], rows_v[r], semg[r])

    def wait_gather(q, r):
        pltpu.make_async_copy(vsrc.at[src_v[q]], rows_v[r], semg[r]).wait()

    def start_scatter(q, r):
        pltpu.async_copy(rows_v[r], acc_sh.at[dst_v[q]], sems[r], add=True)

    def wait_scatter(q, r):
        pltpu.make_async_copy(rows_v[r], acc_sh.at[dst_v[q]], sems[r]).wait()

    for c in range(4):
        start_idx(c, c)

    nouter = NCHUNK // 8

    @pl.loop(0, nouter)
    def _(k):
        i0 = 8 * k
        for m in range(8):
            i = i0 + m
            q = m
            r = m % 4
            qp = (m + 6) % 8
            rp = (m + 2) % 4
            wait_idx(i, q)
            if m >= 4:
                wait_scatter(q, r)
            else:
                @pl.when(k >= 1)
                def _():
                    wait_scatter(q, r)
            start_gather(q, r)
            if m >= 2:
                wait_gather(qp, rp)
                _scale_rows(rows_v[rp], w_v[qp])
                start_scatter(qp, rp)
            else:
                @pl.when(k >= 1)
                def _():
                    wait_gather(qp, rp)
                    _scale_rows(rows_v[rp], w_v[qp])
                    start_scatter(qp, rp)
            if m < 4:
                start_idx(i + 4, m + 4)
            else:
                @pl.when(k < nouter - 1)
                def _():
                    start_idx(i + 4, m - 4)

    for i in (NCHUNK - 2, NCHUNK - 1):
        wait_gather(i % 8, i % 4)
        _scale_rows(rows_v[i % 4], w_v[i % 8])
        start_scatter(i % 8, i % 4)
    for i in range(NCHUNK - 4, NCHUNK):
        wait_scatter(i % 8, i % 4)


def _chain_phase(cid, r0, e0, vp, srcr, dstr, wr, d2r, out1, out2, qout,
                 acc_sh, bufs):
    (src_v, dst_v, w_v, rows_v, semi, semg, sems) = bufs
    _zero_rows(rows_v[0])
    for b in range(RPT // CH):
        pltpu.sync_copy(rows_v[0], acc_sh.at[pl.ds(r0 + b * CH, CH)])
    plsc.subcore_barrier()

    _edge_pass(e0, srcr, dstr, wr, vp.at[cid], acc_sh, bufs)
    plsc.subcore_barrier()

    for b in range(RPT // CH):
        rb = r0 + b * CH
        pltpu.sync_copy(acc_sh.at[pl.ds(rb, CH)], rows_v[0])
        pltpu.sync_copy(rows_v[0], out1.at[cid, pl.ds(rb, CH)])
        pltpu.sync_copy(d2r.at[pl.ds(rb, CH)], w_v[0])
        _scale_rows(rows_v[0], w_v[0])
        pltpu.sync_copy(rows_v[0], qout.at[cid, pl.ds(rb, CH)])
        _zero_rows(rows_v[0])
        pltpu.sync_copy(rows_v[0], acc_sh.at[pl.ds(rb, CH)])
    plsc.subcore_barrier()

    _edge_pass(e0, srcr, dstr, wr, qout.at[cid], acc_sh, bufs)
    plsc.subcore_barrier()

    for b in range(RPT // CH):
        rb = r0 + b * CH
        pltpu.sync_copy(acc_sh.at[pl.ds(rb, CH)], rows_v[0])
        pltpu.sync_copy(rows_v[0], out2.at[cid, pl.ds(rb, CH)])


NIDX = 8
NROW = 4

_SC_SCRATCH = (
    [pltpu.VMEM_SHARED((NP, HALF), jnp.float32)]
    + [pltpu.VMEM((CH,), jnp.int32) for _ in range(NIDX)]
    + [pltpu.VMEM((CH,), jnp.int32) for _ in range(NIDX)]
    + [pltpu.VMEM((CH,), jnp.float32) for _ in range(NIDX)]
    + [pltpu.VMEM((CH, HALF), jnp.float32) for _ in range(NROW)]
    + [pltpu.SemaphoreType.DMA for _ in range(NIDX + 2 * NROW)]
)


def _pack_bufs(scr):
    scr = scr[1:]
    src_v = tuple(scr[0:NIDX])
    dst_v = tuple(scr[NIDX:2 * NIDX])
    w_v = tuple(scr[2 * NIDX:3 * NIDX])
    rows_v = tuple(scr[3 * NIDX:3 * NIDX + NROW])
    sems_all = scr[3 * NIDX + NROW:]
    semi = tuple(sems_all[0:NIDX])
    semg = tuple(sems_all[NIDX:NIDX + NROW])
    sems = tuple(sems_all[NIDX + NROW:NIDX + 2 * NROW])
    return (src_v, dst_v, w_v, rows_v, semi, semg, sems)


_P_TYPE = jax.ShapeDtypeStruct((2, NP, HALF), jnp.float32)
_PT_TYPE = jax.ShapeDtypeStruct((T, 2, NP, HALF), jnp.float32)


@functools.cache
def _build_chain_kernels():
    mesh = _make_mesh()

    @functools.partial(pl.kernel,
                       out_type=[_P_TYPE, _P_TYPE, _P_TYPE], mesh=mesh,
                       compiler_params=_SC_PARAMS, scratch_types=_SC_SCRATCH)
    def chain_one(vp, srcr, dstr, wr, d2r, out1, out2, qout, *scr):
        cid = lax.axis_index("c")
        sid = lax.axis_index("s")
        bufs = _pack_bufs(scr)
        _chain_phase(cid, sid * RPT, sid * EPT, vp, srcr, dstr, wr, d2r,
                     out1, out2, qout, scr[0], bufs)

    @functools.partial(pl.kernel,
                       out_type=[_PT_TYPE, _PT_TYPE, _P_TYPE], mesh=mesh,
                       compiler_params=_SC_PARAMS, scratch_types=_SC_SCRATCH)
    def chain_all(vp, srcr, dstr, wr, d2r, out1, out2, qout, *scr):
        cid = lax.axis_index("c")
        sid = lax.axis_index("s")
        bufs = _pack_bufs(scr)

        @pl.loop(0, T)
        def _(t):
            _chain_phase(cid, sid * RPT, sid * EPT, vp.at[t], srcr.at[t],
                         dstr.at[t], wr.at[t], d2r.at[t], out1.at[t],
                         out2.at[t], qout, scr[0], bufs)
            plsc.subcore_barrier()

    return chain_one, chain_all


def _chain_sc(vp, srcr, dstr, wr, d2r):
    return _build_chain_kernels()[0](vp, srcr, dstr, wr, d2r)[:2]


def _chain_all_sc(vp, srcr, dstr, wr, d2r):
    return _build_chain_kernels()[1](vp, srcr, dstr, wr, d2r)[:2]


def _cat_scaled(p_ref, dv):
    return jnp.concatenate([p_ref[0], p_ref[1]], axis=1) * dv


def _gx_body(x_ref, a1_ref, a2_ref, dv_ref, w_ref, b_ref, o_ref):
    x = x_ref[0]
    dv = dv_ref[0]
    a1 = _cat_scaled(a1_ref[0], dv)
    a2 = _cat_scaled(a2_ref[0], dv)
    basis = (x, a1, a2)
    for g in range(3):
        acc = jnp.broadcast_to(b_ref[g][None, :], (ROW_BLK, D))
        for k in range(3):
            acc = acc + jnp.dot(basis[k], w_ref[g, k],
                                preferred_element_type=jnp.float32)
        o_ref[g, 0] = acc


def _gx_matmuls(x, a1raw, a2raw, dv, w, b):
    grid = (T, N // ROW_BLK)
    return pl.pallas_call(
        _gx_body,
        grid=grid,
        in_specs=[
            pl.BlockSpec((1, ROW_BLK, D), lambda t, i: (t, i, 0)),
            pl.BlockSpec((1, 2, ROW_BLK, HALF), lambda t, i: (t, 0, i, 0)),
            pl.BlockSpec((1, 2, ROW_BLK, HALF), lambda t, i: (t, 0, i, 0)),
            pl.BlockSpec((1, ROW_BLK, 1), lambda t, i: (t, i, 0)),
            pl.BlockSpec((3, 3, D, D), lambda t, i: (0, 0, 0, 0)),
            pl.BlockSpec((3, D), lambda t, i: (0, 0)),
        ],
        out_specs=pl.BlockSpec((3, 1, ROW_BLK, D), lambda t, i: (0, t, i, 0)),
        out_shape=jax.ShapeDtypeStruct((3, T, N, D), jnp.float32),
    )(x, a1raw, a2raw, dv, w, b)


def _ru_body(h_ref, p1_ref, p2_ref, dv_ref, gr_ref, gu_ref, wr_ref, wu_ref,
             rh_ref, rhp_ref, u_ref):
    h = h_ref[...]
    dv = dv_ref[...]
    h1 = _cat_scaled(p1_ref, dv)
    h2 = _cat_scaled(p2_ref, dv)
    pre_r = gr_ref[...]
    pre_u = gu_ref[...]
    basis = (h, h1, h2)
    for k in range(3):
        pre_r = pre_r + jnp.dot(basis[k], wr_ref[k],
                                preferred_element_type=jnp.float32)
        pre_u = pre_u + jnp.dot(basis[k], wu_ref[k],
                                preferred_element_type=jnp.float32)
    rh = jax.nn.sigmoid(pre_r) * h
    rh_ref[...] = rh
    rhp = rh * dv
    rhp_ref[0] = rhp[:, :HALF]
    rhp_ref[1] = rhp[:, HALF:]
    u_ref[...] = jax.nn.sigmoid(pre_u)


def _ru_stage(h, p1, p2, dv, gr, gu, wr, wu):
    grid = (N // ROW_BLK,)
    blk = pl.BlockSpec((ROW_BLK, D), lambda i: (i, 0))
    pblk = pl.BlockSpec((2, ROW_BLK, HALF), lambda i: (0, i, 0))
    dblk = pl.BlockSpec((ROW_BLK, 1), lambda i: (i, 0))
    wblk = pl.BlockSpec((3, D, D), lambda i: (0, 0, 0))
    return pl.pallas_call(
        _ru_body,
        grid=grid,
        in_specs=[blk, pblk, pblk, dblk, blk, blk, wblk, wblk],
        out_specs=[blk, pblk, blk],
        out_shape=[jax.ShapeDtypeStruct((N, D), jnp.float32),
                   jax.ShapeDtypeStruct((2, NP, HALF), jnp.float32),
                   jax.ShapeDtypeStruct((N, D), jnp.float32)],
    )(h, p1, p2, dv, gr, gu, wr, wu)


def _c_body(rh_ref, p1_ref, p2_ref, dv_ref, dvn_ref, gm_ref, u_ref, h_ref,
            wm_ref, h_out, hpn_out):
    dv = dv_ref[...]
    m1 = _cat_scaled(p1_ref, dv)
    m2 = _cat_scaled(p2_ref, dv)
    pre = gm_ref[...]
    basis = (rh_ref[...], m1, m2)
    for k in range(3):
        pre = pre + jnp.dot(basis[k], wm_ref[k],
                            preferred_element_type=jnp.float32)
    c = jnp.tanh(pre)
    u = u_ref[...]
    hn = u * h_ref[...] + (1.0 - u) * c
    h_out[...] = hn
    hp = hn * dvn_ref[...]
    hpn_out[0] = hp[:, :HALF]
    hpn_out[1] = hp[:, HALF:]


def _c_stage(rh, p1, p2, dv, dvn, gm, u, h, wm):
    grid = (N // ROW_BLK,)
    blk = pl.BlockSpec((ROW_BLK, D), lambda i: (i, 0))
    pblk = pl.BlockSpec((2, ROW_BLK, HALF), lambda i: (0, i, 0))
    dblk = pl.BlockSpec((ROW_BLK, 1), lambda i: (i, 0))
    wblk = pl.BlockSpec((3, D, D), lambda i: (0, 0, 0))
    return pl.pallas_call(
        _c_body,
        grid=grid,
        in_specs=[blk, pblk, pblk, dblk, dblk, blk, blk, blk, wblk],
        out_specs=[blk, pblk],
        out_shape=[jax.ShapeDtypeStruct((N, D), jnp.float32),
                   jax.ShapeDtypeStruct((2, NP, HALF), jnp.float32)],
    )(rh, p1, p2, dv, dvn, gm, u, h, wm)


def kernel(x, edge_idx, edge_attr, reset_W, reset_b, update_W, update_b,
           mem_W, mem_b):
    f32 = jnp.float32
    src = edge_idx[:, 0]
    dst = edge_idx[:, 1]
    w = edge_attr

    deg = jax.vmap(lambda s, ww: jnp.zeros((N,), f32).at[s].add(ww))(src, w)
    dinv = jnp.where(deg > 0, jnp.where(deg > 0, deg, 1.0) ** -0.5, 0.0)
    dv = dinv[:, :, None]
    dvn = jnp.roll(dv, -1, axis=0)
    d2p = jnp.pad(dinv * dinv, ((0, 0), (0, NP - N)))

    pad_idx = (jnp.arange(EP - E, dtype=jnp.int32) % N)[None, :]
    pad_idx = jnp.broadcast_to(pad_idx, (T, EP - E))
    srcp = jnp.concatenate([src, pad_idx], axis=1)
    dstp = jnp.concatenate([dst, pad_idx], axis=1)
    wneg = jnp.concatenate([-w, jnp.zeros((T, EP - E), f32)], axis=1)

    def fold(W):
        Wx, Wh = W[:, :D, :], W[:, D:, :]

        def f(Wp):
            return jnp.stack([Wp[0] - Wp[2], Wp[1], 2.0 * Wp[2]])
        return f(Wx), f(Wh)

    Wrx, Wrh = fold(reset_W)
    Wux, Wuh = fold(update_W)
    Wmx, Wmh = fold(mem_W)
    Wx_all = jnp.stack([Wrx, Wux, Wmx])
    b_all = jnp.stack([reset_b, update_b, mem_b])

    xp = x * dv
    xp = jnp.pad(xp, ((0, 0), (0, NP - N), (0, 0)))
    xp = xp.reshape(T, NP, 2, HALF).transpose(0, 2, 1, 3)
    a1raw, a2raw = _chain_all_sc(xp, srcp, dstp, wneg, d2p)
    gx = _gx_matmuls(x, a1raw, a2raw, dv, Wx_all, b_all)

    def step(carry, args):
        h, hp2 = carry
        s, d, w_t, d2_t, dv_t, dvn_t, gr, gu, gm = args
        p1h, p2h = _chain_sc(hp2, s, d, w_t, d2_t)
        rh, rhp2, u = _ru_stage(h, p1h, p2h, dv_t, gr, gu, Wrh, Wuh)
        p1m, p2m = _chain_sc(rhp2, s, d, w_t, d2_t)
        hn, hpn = _c_stage(rh, p1m, p2m, dv_t, dvn_t, gm, u, h, Wmh)
        return (hn, hpn), hn

    h0 = jnp.zeros((N, D), f32)
    hp0 = jnp.zeros((2, NP, HALF), f32)
    (h_fin, _), hs = jax.lax.scan(
        step, (h0, hp0), (srcp, dstp, wneg, d2p, dv, dvn, gx[0], gx[1], gx[2]))
    return (h_fin, hs)
